# Initial kernel scaffold; baseline (speedup 1.0000x reference)
import jax
import jax.numpy as jnp
from jax.experimental import pallas as pl


def kernel(text1, len1, text2, len2, emb, W1, b1, W2, b2):
    raise NotImplementedError("write your pallas kernel here")


# speedup vs baseline: 1.2312x; 1.2312x over previous
import functools

import jax
import jax.numpy as jnp
from jax import lax
from jax.experimental import pallas as pl
from jax.experimental.pallas import tpu as pltpu
from jax.experimental.pallas import tpu_sc as plsc


def _make_sc_pool(B, L, D, nc, ns, interpret=False):
    NW = nc * ns
    assert B % NW == 0
    BPW = B // NW
    S0 = (L // 2 + 7) & ~7
    S1 = L - S0
    assert 0 < S1 <= 128 and S0 <= 128 and S0 % 8 == 0
    NCHUNK = D // 16
    mesh = plsc.VectorSubcoreMesh(
        core_axis_name="c", subcore_axis_name="s",
        num_cores=nc, num_subcores=ns)

    @functools.partial(
        pl.kernel,
        out_type=jax.ShapeDtypeStruct((B, 2 * D), jnp.float32),
        mesh=mesh,
        scratch_types=[
            pltpu.VMEM((BPW * L,), jnp.int32),
            pltpu.VMEM((BPW * L,), jnp.int32),
            pltpu.VMEM((L, D), jnp.float32),
            pltpu.VMEM((BPW, 2 * D), jnp.float32),
            pltpu.SemaphoreType.DMA,
        ],
        compiler_params=pltpu.CompilerParams(use_tc_tiling_on_sc=False),
        interpret=interpret,
    )
    def sc_pool(t1_hbm, t2_hbm, emb_hbm, out_hbm, idx1_v, idx2_v,
                buf_v, out_v, sem):
        wid = lax.axis_index("s") * nc + lax.axis_index("c")
        base = wid * BPW
        pltpu.sync_copy(t1_hbm.at[pl.ds(base * L, BPW * L)], idx1_v)
        pltpu.sync_copy(t2_hbm.at[pl.ds(base * L, BPW * L)], idx2_v)
        for t, idx_v in ((0, idx1_v), (1, idx2_v)):
            def row_body(r, carry, idx_v=idx_v, t=t):
                o0 = pl.multiple_of(r * L, 8)
                o1 = pl.multiple_of(r * L + S0, 8)
                c0 = pltpu.async_copy(
                    emb_hbm.at[idx_v.at[pl.ds(o0, S0)]],
                    buf_v.at[pl.ds(0, S0)], sem)
                c1 = pltpu.async_copy(
                    emb_hbm.at[idx_v.at[pl.ds(o1, S1)]],
                    buf_v.at[pl.ds(S0, S1)], sem)
                c0.wait()
                c1.wait()

                def acc_body(l, accs):
                    return tuple(a + buf_v[l, pl.ds(16 * c, 16)]
                                 for c, a in enumerate(accs))

                z = jnp.zeros((16,), jnp.float32)
                accs = lax.fori_loop(0, L, acc_body, (z,) * NCHUNK)
                for c in range(NCHUNK):
                    out_v[r, pl.ds(t * D + 16 * c, 16)] = accs[c]
                return carry

            lax.fori_loop(0, BPW, row_body, 0)
        pltpu.sync_copy(out_v, out_hbm.at[pl.ds(base, BPW)])

    return sc_pool


def _make_mlp(B, L, D, H, C, BB=512, interpret=False):
    def body(sums_ref, t1_ref, t2_ref, len1_ref, len2_ref, row1_ref,
             W1_ref, b1_ref, W2_ref, b2_ref, logit_ref, prob_ref):
        s = sums_ref[...]
        r1 = row1_ref[...]
        cnt1 = jnp.sum((t1_ref[...] == 1).astype(jnp.float32), axis=1,
                       keepdims=True)
        cnt2 = jnp.sum((t2_ref[...] == 1).astype(jnp.float32), axis=1,
                       keepdims=True)
        e1 = (s[:, :D] - cnt1 * r1) / len1_ref[...]
        e2 = (s[:, D:] - cnt2 * r1) / len2_ref[...]
        x = jnp.concatenate([e1, e2, jnp.abs(e1 - e2), e1 * e2], axis=1)
        h = lax.dot_general(x, W1_ref[...], (((1,), (1,)), ((), ())),
                            precision=lax.Precision.HIGHEST,
                            preferred_element_type=jnp.float32)
        h = jnp.maximum(h + b1_ref[...], 0.0)
        logit = lax.dot_general(h, W2_ref[...], (((1,), (1,)), ((), ())),
                                precision=lax.Precision.HIGHEST,
                                preferred_element_type=jnp.float32)
        logit = logit + b2_ref[...]
        m = jnp.max(logit, axis=1, keepdims=True)
        ex = jnp.exp(logit - m)
        prob = ex / jnp.sum(ex, axis=1, keepdims=True)
        logit_ref[...] = logit
        prob_ref[...] = prob

    grid = (B // BB,)
    blk = lambda i: (i, 0)
    fixed = lambda i: (0, 0)
    return pl.pallas_call(
        body,
        grid=grid,
        in_specs=[
            pl.BlockSpec((BB, 2 * D), blk),
            pl.BlockSpec((BB, L), blk),
            pl.BlockSpec((BB, L), blk),
            pl.BlockSpec((BB, 1), blk),
            pl.BlockSpec((BB, 1), blk),
            pl.BlockSpec((1, D), fixed),
            pl.BlockSpec((H, 4 * D), fixed),
            pl.BlockSpec((1, H), fixed),
            pl.BlockSpec((C, H), fixed),
            pl.BlockSpec((1, C), fixed),
        ],
        out_specs=[
            pl.BlockSpec((BB, C), blk),
            pl.BlockSpec((BB, C), blk),
        ],
        out_shape=[
            jax.ShapeDtypeStruct((B, C), jnp.float32),
            jax.ShapeDtypeStruct((B, C), jnp.float32),
        ],
        interpret=interpret,
    )


def kernel(text1, len1, text2, len2, emb, W1, b1, W2, b2):
    B, L = text1.shape
    D = emb.shape[1]
    H = W1.shape[0]
    C = W2.shape[0]
    t1 = jnp.asarray(text1, jnp.int32)
    t2 = jnp.asarray(text2, jnp.int32)
    info = plsc.get_sparse_core_info()
    sc_pool = _make_sc_pool(B, L, D, info.num_cores, info.num_subcores)
    sums = sc_pool(t1.reshape(-1), t2.reshape(-1), emb)
    row1 = lax.slice(emb, (1, 0), (2, D))
    mlp = _make_mlp(B, L, D, H, C)
    logit, prob = mlp(sums, t1, t2, len1.reshape(B, 1), len2.reshape(B, 1),
                      row1, W1, b1.reshape(1, H), W2, b2.reshape(1, C))
    return (logit, prob)


# speedup vs baseline: 1.4518x; 1.1792x over previous
import functools

import jax
import jax.numpy as jnp
from jax import lax
from jax.experimental import pallas as pl
from jax.experimental.pallas import tpu as pltpu
from jax.experimental.pallas import tpu_sc as plsc

NBUF = 4
KOFF = 2


def _make_sc_pool(B, L, D, nc, ns, interpret=False):
    NW = nc * ns
    assert B % NW == 0
    BPW = B // NW
    assert (BPW * L) % 128 == 0
    NCH = BPW * L // 128
    NGRP = (NCH + KOFF + NBUF - 1) // NBUF
    mesh = plsc.VectorSubcoreMesh(
        core_axis_name="c", subcore_axis_name="s",
        num_cores=nc, num_subcores=ns)

    @functools.partial(
        pl.kernel,
        out_type=jax.ShapeDtypeStruct((2 * B, D), jnp.float32),
        mesh=mesh,
        scratch_types=[
            pltpu.VMEM((BPW * L,), jnp.int32),
            pltpu.VMEM((NCH, 128), jnp.int32),
            pltpu.VMEM_SHARED((ns * BPW, D), jnp.float32),
        ] + [pltpu.VMEM((128, D), jnp.float32) for _ in range(NBUF)]
          + [pltpu.SemaphoreType.DMA for _ in range(2 * NBUF)],
        compiler_params=pltpu.CompilerParams(use_tc_tiling_on_sc=False),
        interpret=interpret,
    )
    def sc_pool(t1_hbm, t2_hbm, emb_hbm, didx_hbm, out_hbm,
                idx_v, didx_v, acc_sh, *bufs_and_sems):
        bufs = bufs_and_sems[:NBUF]
        g_sem = bufs_and_sems[NBUF:2 * NBUF]
        s_sem = bufs_and_sems[2 * NBUF:]
        s_idx = lax.axis_index("s")
        wid = s_idx * nc + lax.axis_index("c")
        base = wid * BPW
        lbase = s_idx * BPW

        pltpu.sync_copy(didx_hbm, didx_v)

        def add_base(j, carry):
            for m in range(8):
                didx_v[j, pl.ds(16 * m, 16)] = (
                    didx_v[j, pl.ds(16 * m, 16)]
                    + jnp.full((16,), 1, jnp.int32) * lbase)
            return carry

        lax.fori_loop(0, NCH, add_base, 0)

        for t, t_hbm in ((0, t1_hbm), (1, t2_hbm)):
            pltpu.sync_copy(t_hbm.at[pl.ds(base * L, BPW * L)], idx_v)

            def zero_body(i, carry):
                for m in range(D // 16):
                    bufs[0][i, pl.ds(16 * m, 16)] = jnp.zeros((16,),
                                                              jnp.float32)
                return carry

            lax.fori_loop(0, BPW, zero_body, 0)
            pltpu.sync_copy(
                bufs[0], acc_sh.at[pl.ds(pl.multiple_of(lbase, 8), BPW)])

            def gather_src(j):
                off = pl.multiple_of(j * 128, 8)
                return emb_hbm.at[idx_v.at[pl.ds(off, 128)]]

            def grp(g, carry):
                for b in range(NBUF):
                    j = g * NBUF + b
                    m = j - KOFF
                    sb = (b + NBUF - KOFF) % NBUF

                    @pl.when(jnp.logical_and(m >= 0, m < NCH))
                    def _scat(m=m, sb=sb):
                        pltpu.make_async_copy(
                            gather_src(m), bufs[sb], g_sem[sb]).wait()
                        pltpu.async_copy(
                            bufs[sb], acc_sh.at[didx_v.at[m]], s_sem[sb],
                            add=True)

                    @pl.when(j < NCH)
                    def _gath(j=j, b=b):
                        @pl.when(j >= NBUF)
                        def _drain(j=j, b=b):
                            pltpu.make_async_copy(
                                bufs[b], acc_sh.at[didx_v.at[j - NBUF]],
                                s_sem[b]).wait()

                        pltpu.async_copy(gather_src(j), bufs[b], g_sem[b])
                return carry

            lax.fori_loop(0, NGRP, grp, 0)
            for b in range(NBUF):
                m = NCH - NBUF + b
                pltpu.make_async_copy(
                    bufs[m % NBUF], acc_sh.at[didx_v.at[m]],
                    s_sem[m % NBUF]).wait()
            pltpu.sync_copy(
                acc_sh.at[pl.ds(pl.multiple_of(lbase, 8), BPW)],
                out_hbm.at[pl.ds(pl.multiple_of(t * B + base, 8), BPW)])

    return sc_pool


def _make_mlp(B, L, D, H, C, BB=512, interpret=False):
    def body(s1_ref, s2_ref, t1_ref, t2_ref, len1_ref, len2_ref, row1_ref,
             W1_ref, b1_ref, W2_ref, b2_ref, logit_ref, prob_ref):
        r1 = row1_ref[...]
        cnt1 = jnp.sum((t1_ref[...] == 1).astype(jnp.float32), axis=1,
                       keepdims=True)
        cnt2 = jnp.sum((t2_ref[...] == 1).astype(jnp.float32), axis=1,
                       keepdims=True)
        e1 = (s1_ref[...] - cnt1 * r1) / len1_ref[...]
        e2 = (s2_ref[...] - cnt2 * r1) / len2_ref[...]
        x = jnp.concatenate([e1, e2, jnp.abs(e1 - e2), e1 * e2], axis=1)
        h = lax.dot_general(x, W1_ref[...], (((1,), (1,)), ((), ())),
                            precision=lax.Precision.HIGHEST,
                            preferred_element_type=jnp.float32)
        h = jnp.maximum(h + b1_ref[...], 0.0)
        logit = lax.dot_general(h, W2_ref[...], (((1,), (1,)), ((), ())),
                                precision=lax.Precision.HIGHEST,
                                preferred_element_type=jnp.float32)
        logit = logit + b2_ref[...]
        m = jnp.max(logit, axis=1, keepdims=True)
        ex = jnp.exp(logit - m)
        prob = ex / jnp.sum(ex, axis=1, keepdims=True)
        logit_ref[...] = logit
        prob_ref[...] = prob

    grid = (B // BB,)
    blk = lambda i: (i, 0)
    fixed = lambda i: (0, 0)
    return pl.pallas_call(
        body,
        grid=grid,
        in_specs=[
            pl.BlockSpec((BB, D), blk),
            pl.BlockSpec((BB, D), blk),
            pl.BlockSpec((BB, L), blk),
            pl.BlockSpec((BB, L), blk),
            pl.BlockSpec((BB, 1), blk),
            pl.BlockSpec((BB, 1), blk),
            pl.BlockSpec((1, D), fixed),
            pl.BlockSpec((H, 4 * D), fixed),
            pl.BlockSpec((1, H), fixed),
            pl.BlockSpec((C, H), fixed),
            pl.BlockSpec((1, C), fixed),
        ],
        out_specs=[
            pl.BlockSpec((BB, C), blk),
            pl.BlockSpec((BB, C), blk),
        ],
        out_shape=[
            jax.ShapeDtypeStruct((B, C), jnp.float32),
            jax.ShapeDtypeStruct((B, C), jnp.float32),
        ],
        interpret=interpret,
    )


def kernel(text1, len1, text2, len2, emb, W1, b1, W2, b2):
    B, L = text1.shape
    D = emb.shape[1]
    H = W1.shape[0]
    C = W2.shape[0]
    t1 = jnp.asarray(text1, jnp.int32)
    t2 = jnp.asarray(text2, jnp.int32)
    info = plsc.get_sparse_core_info()
    nc, ns = info.num_cores, info.num_subcores
    BPW = B // (nc * ns)
    didx = (jnp.arange(BPW * L, dtype=jnp.int32) // L).reshape(-1, 128)
    sc_pool = _make_sc_pool(B, L, D, nc, ns)
    sums = sc_pool(t1.reshape(-1), t2.reshape(-1), emb, didx)
    s1 = lax.slice(sums, (0, 0), (B, D))
    s2 = lax.slice(sums, (B, 0), (2 * B, D))
    row1 = lax.slice(emb, (1, 0), (2, D))
    mlp = _make_mlp(B, L, D, H, C)
    logit, prob = mlp(s1, s2, t1, t2, len1.reshape(B, 1), len2.reshape(B, 1),
                      row1, W1, b1.reshape(1, H), W2, b2.reshape(1, C))
    return (logit, prob)
